# baseline (speedup 1.0000x reference)
import jax
import jax.numpy as jnp
from jax import lax
from jax.experimental import pallas as pl
from jax.experimental.pallas import tpu as pltpu
from jax.experimental.pallas import tpu_sc as plsc

_DMODEL = 1024
_B = 4
_T = 2048

_NC = 2
_NS = 16
_NW = _NC * _NS
_TPW = _T // _NW
_CHUNK = 16
_NCH = _TPW // _CHUNK
_NBUF = 3
_LANES = 16
_VPR = _DMODEL // _LANES
_NIT = _NCH * _B


def _emb_body(x_hbm, tok_hbm, pos_hbm, out_hbm,
              idx_v, pos_v, rows_v, gsems, wsems, psem, isem):
    wid = lax.axis_index("s") * _NC + lax.axis_index("c")
    tb = wid * _TPW

    icps = [pltpu.async_copy(x_hbm.at[pl.ds(b * _T + tb, _TPW)],
                             idx_v.at[b], isem) for b in range(_B)]
    pos_cp = pltpu.async_copy(pos_hbm.at[pl.ds(tb, _TPW)], pos_v, psem)
    for cp in icps:
        cp.wait()

    def chunk_cb(i):
        return i // _B, i % _B

    def start_gather(i, p):
        c, b = chunk_cb(i)
        return pltpu.async_copy(
            tok_hbm.at[idx_v.at[b, pl.ds(c * _CHUNK, _CHUNK)]],
            rows_v[p], gsems[p])

    def start_write(i, p):
        c, b = chunk_cb(i)
        base = b * _T + tb + c * _CHUNK
        return pltpu.async_copy(rows_v[p], out_hbm.at[pl.ds(base, _CHUNK)],
                                wsems[p])

    g = [None] * _NBUF
    w = [None] * _NBUF
    for j in range(_NBUF - 1):
        g[j] = start_gather(j, j)
    pos_cp.wait()

    for i in range(_NIT):
        p = i % _NBUF
        nxt = i + _NBUF - 1
        if nxt < _NIT:
            q = nxt % _NBUF
            if w[q] is not None:
                w[q].wait()
                w[q] = None
            g[q] = start_gather(nxt, q)
        g[p].wait()

        c, _ = chunk_cb(i)
        prow = c * _CHUNK
        buf = rows_v[p]

        def add_col(j, carry, prow=prow, buf=buf):
            col = pl.ds(j * _LANES, _LANES)
            for r in range(_CHUNK):
                plsc.addupdate(buf.at[r, col], pos_v[prow + r, col])
            return carry

        lax.fori_loop(0, _VPR, add_col, 0)
        w[p] = start_write(i, p)

    for p in range(_NBUF):
        if w[p] is not None:
            w[p].wait()


@jax.jit
def kernel(x, tokemb, posemb):
    b, t = x.shape
    mesh = plsc.VectorSubcoreMesh(core_axis_name="c", subcore_axis_name="s")
    out = pl.kernel(
        _emb_body,
        out_type=jax.ShapeDtypeStruct((b * t, _DMODEL), jnp.float32),
        mesh=mesh,
        scratch_types=[
            pltpu.VMEM((_B, _TPW), jnp.int32),
            pltpu.VMEM((_TPW, _DMODEL), jnp.float32),
            [pltpu.VMEM((_CHUNK, _DMODEL), jnp.float32)] * _NBUF,
            [pltpu.SemaphoreType.DMA] * _NBUF,
            [pltpu.SemaphoreType.DMA] * _NBUF,
            pltpu.SemaphoreType.DMA,
            pltpu.SemaphoreType.DMA,
        ],
    )(x.reshape(b * t).astype(jnp.int32), tokemb, posemb)
    return out.reshape(b, t, _DMODEL)


# speedup vs baseline: 1.1631x; 1.1631x over previous
import jax
import jax.numpy as jnp
from jax import lax
from jax.experimental import pallas as pl
from jax.experimental.pallas import tpu as pltpu
from jax.experimental.pallas import tpu_sc as plsc

_DMODEL = 1024
_B = 4
_T = 2048

_NC = 2
_NS = 16
_NW = _NC * _NS
_TPW = _T // _NW
_CHUNK = 16
_NCH = _TPW // _CHUNK
_NBUF = 3
_LANES = 16
_VPR = _DMODEL // _LANES
_NIT = _NCH * _B


def _emb_body(x_hbm, tok_hbm, pos_hbm, out_hbm,
              idx_v, pos_v, rows_v, gsems, wsems, psem, isem):
    wid = lax.axis_index("s") * _NC + lax.axis_index("c")
    tb = wid * _TPW

    icps = [pltpu.async_copy(x_hbm.at[pl.ds(b * _T + tb, _TPW)],
                             idx_v.at[b], isem) for b in range(_B)]
    pos_cp = pltpu.async_copy(pos_hbm.at[pl.ds(tb, _TPW)], pos_v, psem)
    for cp in icps:
        cp.wait()

    def chunk_cb(i):
        return i // _B, i % _B

    def start_gather(i, p):
        c, b = chunk_cb(i)
        return pltpu.async_copy(
            tok_hbm.at[idx_v.at[b, pl.ds(c * _CHUNK, _CHUNK)]],
            rows_v[p], gsems[p])

    def start_write(i, p):
        c, b = chunk_cb(i)
        base = b * _T + tb + c * _CHUNK
        return pltpu.async_copy(rows_v[p], out_hbm.at[pl.ds(base, _CHUNK)],
                                wsems[p])

    g = [None] * _NBUF
    w = [None] * _NBUF
    for j in range(_NBUF - 1):
        g[j] = start_gather(j, j)
    pos_cp.wait()

    for i in range(_NIT):
        p = i % _NBUF
        nxt = i + _NBUF - 1
        if nxt < _NIT:
            q = nxt % _NBUF
            if w[q] is not None:
                w[q].wait()
                w[q] = None
            g[q] = start_gather(nxt, q)
        with jax.named_scope("gwait"):
            g[p].wait()

        c, _ = chunk_cb(i)
        prow = c * _CHUNK
        buf = rows_v[p]

        def add_col(j, carry, prow=prow, buf=buf):
            col = pl.ds(j * _LANES, _LANES)
            for r in range(_CHUNK):
                buf[r, col] = buf[r, col] + pos_v[prow + r, col]
            return carry

        with jax.named_scope("add"):
            lax.fori_loop(0, _VPR, add_col, 0)
        w[p] = start_write(i, p)

    with jax.named_scope("wdrain"):
        for p in range(_NBUF):
            if w[p] is not None:
                w[p].wait()


@jax.jit
def kernel(x, tokemb, posemb):
    b, t = x.shape
    mesh = plsc.VectorSubcoreMesh(core_axis_name="c", subcore_axis_name="s")
    out = pl.kernel(
        _emb_body,
        out_type=jax.ShapeDtypeStruct((b * t, _DMODEL), jnp.float32),
        mesh=mesh,
        scratch_types=[
            pltpu.VMEM((_B, _TPW), jnp.int32),
            pltpu.VMEM((_TPW, _DMODEL), jnp.float32),
            [pltpu.VMEM((_CHUNK, _DMODEL), jnp.float32)] * _NBUF,
            [pltpu.SemaphoreType.DMA] * _NBUF,
            [pltpu.SemaphoreType.DMA] * _NBUF,
            pltpu.SemaphoreType.DMA,
            pltpu.SemaphoreType.DMA,
        ],
    )(x.reshape(b * t).astype(jnp.int32), tokemb, posemb)
    return out.reshape(b, t, _DMODEL)


# speedup vs baseline: 1.1984x; 1.0303x over previous
import jax
import jax.numpy as jnp
from jax import lax
from jax.experimental import pallas as pl
from jax.experimental.pallas import tpu as pltpu
from jax.experimental.pallas import tpu_sc as plsc

_DMODEL = 1024
_B = 4
_T = 2048

_NC = 2
_NS = 16
_NW = _NC * _NS
_TPW = _T // _NW
_CHUNK = 8
_NSET = _TPW // _CHUNK
_PHALF = _TPW // 2
_LANES = 16
_VPR = _DMODEL // _LANES


def _emb_body(x_hbm, tok_hbm, pos_hbm, out_hbm,
              idx_v, pos_v, rows_v, gsems, wsems, psem, isem):
    wid = lax.axis_index("s") * _NC + lax.axis_index("c")
    tb = wid * _TPW

    icps = [pltpu.async_copy(x_hbm.at[pl.ds(b * _T + tb, _TPW)],
                             idx_v.at[b], isem) for b in range(_B)]
    pos_cp = pltpu.async_copy(pos_hbm.at[pl.ds(tb, _PHALF)], pos_v, psem)
    for cp in icps:
        cp.wait()

    def start_gather(s, b):
        p = (s % 2) * _B + b
        return pltpu.async_copy(
            tok_hbm.at[idx_v.at[b, pl.ds(s * _CHUNK, _CHUNK)]],
            rows_v[p], gsems[p])

    def start_write(s, b):
        p = (s % 2) * _B + b
        base = b * _T + tb + s * _CHUNK
        return pltpu.async_copy(rows_v[p], out_hbm.at[pl.ds(base, _CHUNK)],
                                wsems[p])

    g = [None] * (2 * _B)
    w = [None] * (2 * _B)
    for s in range(2):
        for b in range(_B):
            g[s * _B + b] = start_gather(s, b)
    pos_cp.wait()

    mid = _NSET // 2
    for s in range(_NSET):
        pi = (s % 2) * _B
        if s + 2 < _NSET:
            for b in range(_B):
                if w[pi + b] is not None:
                    w[pi + b].wait()
                    w[pi + b] = None
            for b in range(_B):
                g[pi + b] = start_gather(s + 2, b)
        if s == mid:
            pos_cp = pltpu.async_copy(pos_hbm.at[pl.ds(tb + _PHALF, _PHALF)],
                                      pos_v, psem)
        for b in range(_B):
            g[pi + b].wait()
        if s == mid:
            pos_cp.wait()

        prow = (s % (_PHALF // _CHUNK)) * _CHUNK
        bufs = [rows_v[pi + b] for b in range(_B)]

        def add_col(j, carry, prow=prow, bufs=bufs):
            col = pl.ds(j * _LANES, _LANES)
            pvs = [pos_v[prow + r, col] for r in range(_CHUNK)]
            for buf in bufs:
                for r in range(_CHUNK):
                    buf[r, col] = buf[r, col] + pvs[r]
            return carry

        lax.fori_loop(0, _VPR, add_col, 0)
        for b in range(_B):
            w[pi + b] = start_write(s, b)

    for p in range(2 * _B):
        if w[p] is not None:
            w[p].wait()


@jax.jit
def kernel(x, tokemb, posemb):
    b, t = x.shape
    mesh = plsc.VectorSubcoreMesh(core_axis_name="c", subcore_axis_name="s")
    out = pl.kernel(
        _emb_body,
        out_type=jax.ShapeDtypeStruct((b * t, _DMODEL), jnp.float32),
        mesh=mesh,
        scratch_types=[
            pltpu.VMEM((_B, _TPW), jnp.int32),
            pltpu.VMEM((_PHALF, _DMODEL), jnp.float32),
            [pltpu.VMEM((_CHUNK, _DMODEL), jnp.float32)] * (2 * _B),
            [pltpu.SemaphoreType.DMA] * (2 * _B),
            [pltpu.SemaphoreType.DMA] * (2 * _B),
            pltpu.SemaphoreType.DMA,
            pltpu.SemaphoreType.DMA,
        ],
    )(x.reshape(b * t).astype(jnp.int32), tokemb, posemb)
    return out.reshape(b, t, _DMODEL)


# speedup vs baseline: 1.3053x; 1.0892x over previous
import jax
import jax.numpy as jnp
from jax import lax
from jax.experimental import pallas as pl
from jax.experimental.pallas import tpu as pltpu
from jax.experimental.pallas import tpu_sc as plsc

_DMODEL = 1024
_B = 4
_T = 2048

_NC = 2
_NS = 16
_NW = _NC * _NS
_TPW = _T // _NW
_CHUNK = 8
_NSET = _TPW // _CHUNK
_NGRP = 3
_LANES = 16
_VPR = _DMODEL // _LANES


def _emb_body(x_hbm, tok_hbm, pos_hbm, out_hbm,
              idx_v, pos_v, rows_v, gsems, wsems, psems, isem):
    wid = lax.axis_index("s") * _NC + lax.axis_index("c")
    tb = wid * _TPW

    icps = [pltpu.async_copy(x_hbm.at[pl.ds(b * _T + tb, _TPW)],
                             idx_v.at[b], isem) for b in range(_B)]

    def start_pos(s):
        h = s % 2
        return pltpu.async_copy(
            pos_hbm.at[pl.ds(tb + s * _CHUNK, _CHUNK)], pos_v.at[h], psems[h])

    def start_gather(s, b):
        p = (s % _NGRP) * _B + b
        return pltpu.async_copy(
            tok_hbm.at[idx_v.at[b, pl.ds(s * _CHUNK, _CHUNK)]],
            rows_v[p], gsems[p])

    def start_write(s, b):
        p = (s % _NGRP) * _B + b
        base = b * _T + tb + s * _CHUNK
        return pltpu.async_copy(rows_v[p], out_hbm.at[pl.ds(base, _CHUNK)],
                                wsems[p])

    pcp = [start_pos(0), start_pos(1)]
    for cp in icps:
        cp.wait()
    g = [None] * (_NGRP * _B)
    w = [None] * (_NGRP * _B)
    for s in range(2):
        for b in range(_B):
            g[s * _B + b] = start_gather(s, b)

    for s in range(_NSET):
        pi = (s % _NGRP) * _B
        h = s % 2
        for b in range(_B):
            g[pi + b].wait()
        pcp[h].wait()

        bufs = [rows_v[pi + b] for b in range(_B)]

        def add_col(j, carry, h=h, bufs=bufs):
            col = pl.ds(j * _LANES, _LANES)
            pvs = [pos_v[h, r, col] for r in range(_CHUNK)]
            for buf in bufs:
                for r in range(_CHUNK):
                    buf[r, col] = buf[r, col] + pvs[r]
            return carry

        lax.fori_loop(0, _VPR, add_col, 0)
        for b in range(_B):
            w[pi + b] = start_write(s, b)

        if s + 2 < _NSET:
            pcp[h] = start_pos(s + 2)
            qi = ((s + 2) % _NGRP) * _B
            for b in range(_B):
                if w[qi + b] is not None:
                    w[qi + b].wait()
            for b in range(_B):
                g[qi + b] = start_gather(s + 2, b)

    for p in range(_NGRP * _B):
        if w[p] is not None:
            w[p].wait()


@jax.jit
def kernel(x, tokemb, posemb):
    b, t = x.shape
    mesh = plsc.VectorSubcoreMesh(core_axis_name="c", subcore_axis_name="s")
    out = pl.kernel(
        _emb_body,
        out_type=jax.ShapeDtypeStruct((b * t, _DMODEL), jnp.float32),
        mesh=mesh,
        scratch_types=[
            pltpu.VMEM((_B, _TPW), jnp.int32),
            pltpu.VMEM((2, _CHUNK, _DMODEL), jnp.float32),
            [pltpu.VMEM((_CHUNK, _DMODEL), jnp.float32)] * (_NGRP * _B),
            [pltpu.SemaphoreType.DMA] * (_NGRP * _B),
            [pltpu.SemaphoreType.DMA] * (_NGRP * _B),
            [pltpu.SemaphoreType.DMA] * 2,
            pltpu.SemaphoreType.DMA,
        ],
    )(x.reshape(b * t).astype(jnp.int32), tokemb, posemb)
    return out.reshape(b, t, _DMODEL)


# speedup vs baseline: 1.3054x; 1.0001x over previous
import jax
import jax.numpy as jnp
from jax import lax
from jax.experimental import pallas as pl
from jax.experimental.pallas import tpu as pltpu
from jax.experimental.pallas import tpu_sc as plsc

_DMODEL = 1024
_B = 4
_T = 2048

_NC = 2
_NS = 16
_NW = _NC * _NS
_TPW = _T // _NW
_CHUNK = 8
_NSET = _TPW // _CHUNK
_NGRP = 3
_LANES = 16
_VPR = _DMODEL // _LANES


def _emb_body(x_hbm, tok_hbm, pos_hbm, out_hbm,
              idx_v, pos_v, rows_v, gsems, wsems, psems, isem):
    wid = lax.axis_index("s") * _NC + lax.axis_index("c")
    tb = wid * _TPW

    icps = [pltpu.async_copy(x_hbm.at[pl.ds(b * _T + tb, _TPW)],
                             idx_v.at[b], isem) for b in range(_B)]

    def start_pos(s):
        h = s % 2
        return pltpu.async_copy(
            pos_hbm.at[pl.ds(tb + s * _CHUNK, _CHUNK)], pos_v.at[h], psems[h])

    def start_gather(s, b):
        p = (s % _NGRP) * _B + b
        return pltpu.async_copy(
            tok_hbm.at[idx_v.at[b, pl.ds(s * _CHUNK, _CHUNK)]],
            rows_v[p], gsems[p])

    def start_write(s, b):
        p = (s % _NGRP) * _B + b
        base = b * _T + tb + s * _CHUNK
        return pltpu.async_copy(rows_v[p], out_hbm.at[pl.ds(base, _CHUNK)],
                                wsems[p])

    pcp = [start_pos(0), start_pos(1)]
    for cp in icps:
        cp.wait()
    g = [None] * (_NGRP * _B)
    w = [None] * (_NGRP * _B)
    for s in range(2):
        for b in range(_B):
            g[s * _B + b] = start_gather(s, b)

    for s in range(_NSET):
        pi = (s % _NGRP) * _B
        h = s % 2
        with jax.named_scope("gwait"):
            for b in range(_B):
                g[pi + b].wait()
            pcp[h].wait()

        bufs = [rows_v[pi + b] for b in range(_B)]

        def add_col(j, carry, h=h, bufs=bufs):
            col = pl.ds(j * _LANES, _LANES)
            pvs = [pos_v[h, r, col] for r in range(_CHUNK)]
            for buf in bufs:
                for r in range(_CHUNK):
                    buf[r, col] = buf[r, col] + pvs[r]
            return carry

        with jax.named_scope("add"):
            lax.fori_loop(0, _VPR, add_col, 0)
        for b in range(_B):
            w[pi + b] = start_write(s, b)

        if s + 2 < _NSET:
            pcp[h] = start_pos(s + 2)
            qi = ((s + 2) % _NGRP) * _B
            with jax.named_scope("wwait"):
                for b in range(_B):
                    if w[qi + b] is not None:
                        w[qi + b].wait()
            for b in range(_B):
                g[qi + b] = start_gather(s + 2, b)

    for p in range(_NGRP * _B):
        if w[p] is not None:
            w[p].wait()


@jax.jit
def kernel(x, tokemb, posemb):
    b, t = x.shape
    mesh = plsc.VectorSubcoreMesh(core_axis_name="c", subcore_axis_name="s")
    out = pl.kernel(
        _emb_body,
        out_type=jax.ShapeDtypeStruct((b * t, _DMODEL), jnp.float32),
        mesh=mesh,
        scratch_types=[
            pltpu.VMEM((_B, _TPW), jnp.int32),
            pltpu.VMEM((2, _CHUNK, _DMODEL), jnp.float32),
            [pltpu.VMEM((_CHUNK, _DMODEL), jnp.float32)] * (_NGRP * _B),
            [pltpu.SemaphoreType.DMA] * (_NGRP * _B),
            [pltpu.SemaphoreType.DMA] * (_NGRP * _B),
            [pltpu.SemaphoreType.DMA] * 2,
            pltpu.SemaphoreType.DMA,
        ],
    )(x.reshape(b * t).astype(jnp.int32), tokemb, posemb)
    return out.reshape(b, t, _DMODEL)
